# baseline (speedup 1.0000x reference)
import functools

import jax
import jax.numpy as jnp
from jax import lax
from jax.experimental import pallas as pl
from jax.experimental.pallas import tpu as pltpu
from jax.experimental.pallas import tpu_sc as plsc

NUM_LATENT_CODES = 100000
LATENT_CODE_DIM = 64
LANES = 16
CHUNK = 128
PAIR = 2 * LATENT_CODE_DIM


@functools.lru_cache(maxsize=None)
def _build(batch):
    info = plsc.get_sparse_core_info()
    num_workers = info.num_cores * info.num_subcores
    rows_per_w = batch // num_workers
    n_chunks = rows_per_w // CHUNK
    n_groups = CHUNK // LANES
    scale = float(NUM_LATENT_CODES - 1)

    mesh = plsc.VectorSubcoreMesh(core_axis_name="c", subcore_axis_name="s")

    @functools.partial(
        pl.kernel,
        mesh=mesh,
        out_type=jax.ShapeDtypeStruct((batch, LATENT_CODE_DIM), jnp.float32),
        scratch_types=[
            pltpu.VMEM((CHUNK, LATENT_CODE_DIM), jnp.float32),
            pltpu.VMEM((CHUNK, PAIR), jnp.float32),
            pltpu.VMEM((CHUNK, PAIR), jnp.float32),
            pltpu.VMEM((CHUNK,), jnp.float32),
            pltpu.VMEM((CHUNK,), jnp.int32),
            pltpu.VMEM((CHUNK,), jnp.int32),
            pltpu.VMEM((CHUNK,), jnp.int32),
            pltpu.VMEM((CHUNK,), jnp.int32),
            pltpu.VMEM((CHUNK,), jnp.float32),
            pltpu.VMEM((CHUNK,), jnp.float32),
            pltpu.SemaphoreType.DMA,
            pltpu.SemaphoreType.DMA,
        ],
    )
    def body(xt_hbm, table_hbm, out_hbm, s_v, f_v, c_v, t_v, if_v, ic_v,
             of_v, oc_v, w1_v, w2_v, sem1, sem2):
        wid = lax.axis_index("s") * info.num_cores + lax.axis_index("c")
        base_w = wid * rows_per_w

        for ch in range(n_chunks):
            row0 = base_w + ch * CHUNK

            pltpu.sync_copy(xt_hbm.at[pl.ds(row0, CHUNK)], t_v)

            def phase_idx(g, carry):
                sl = pl.ds(g * LANES, LANES)
                t = t_v[sl]
                t = jnp.minimum(jnp.maximum(t, 0.0), 1.0)
                idx = t * scale
                fl = idx.astype(jnp.int32)
                flf = fl.astype(jnp.float32)
                cl = fl + jnp.sign(idx - flf).astype(jnp.int32)
                if_v[sl] = lax.shift_right_logical(fl, 1)
                ic_v[sl] = lax.shift_right_logical(cl, 1)
                of_v[sl] = (fl & 1) * LATENT_CODE_DIM
                oc_v[sl] = (cl & 1) * LATENT_CODE_DIM
                w1_v[sl] = 1.0 - (idx - flf)
                w2_v[sl] = flf
                return carry

            lax.fori_loop(0, n_groups, phase_idx, 0)

            cp_f = pltpu.async_copy(table_hbm.at[if_v], f_v, sem1)
            cp_c = pltpu.async_copy(table_hbm.at[ic_v], c_v, sem2)
            cp_f.wait()
            cp_c.wait()

            def phase_blend(g, carry):
                sl16 = pl.ds(g * LANES, LANES)
                w1g = w1_v[sl16]
                w2g = w2_v[sl16]
                ofg = of_v[sl16]
                ocg = oc_v[sl16]
                for j in range(LANES):
                    r = g * LANES + j
                    w1 = w1g[j]
                    w2 = w2g[j]
                    pof = ofg[j]
                    poc = ocg[j]
                    for k in range(LATENT_CODE_DIM // LANES):
                        fsl = pl.ds(pof + k * LANES, LANES)
                        csl = pl.ds(poc + k * LANES, LANES)
                        osl = pl.ds(k * LANES, LANES)
                        s_v[r, osl] = w1 * f_v[r, fsl] + w2 * c_v[r, csl]
                return carry

            lax.fori_loop(0, n_groups, phase_blend, 0)

            pltpu.sync_copy(s_v, out_hbm.at[pl.ds(row0, CHUNK), :])

    return body


def kernel(x, latent_codes):
    batch, d_in = x.shape
    table_pairs = latent_codes.reshape(NUM_LATENT_CODES // 2, PAIR)
    sampled = _build(batch)(x[:, -1], table_pairs)
    return jnp.concatenate((x[:, : d_in - 1], sampled), axis=1)


# speedup vs baseline: 1.0288x; 1.0288x over previous
import functools

import jax
import jax.numpy as jnp
from jax import lax
from jax.experimental import pallas as pl
from jax.experimental.pallas import tpu as pltpu
from jax.experimental.pallas import tpu_sc as plsc

NUM_LATENT_CODES = 100000
LATENT_CODE_DIM = 64
LANES = 16
CHUNK = 128
PAIR = 2 * LATENT_CODE_DIM


@functools.lru_cache(maxsize=None)
def _build(batch, d_in):
    d_out = d_in - 1 + LATENT_CODE_DIM
    info = plsc.get_sparse_core_info()
    num_workers = info.num_cores * info.num_subcores
    rows_per_w = batch // num_workers
    n_chunks = rows_per_w // CHUNK
    n_groups = CHUNK // LANES
    scale = float(NUM_LATENT_CODES - 1)

    mesh = plsc.VectorSubcoreMesh(core_axis_name="c", subcore_axis_name="s")

    @functools.partial(
        pl.kernel,
        mesh=mesh,
        out_type=jax.ShapeDtypeStruct((batch, d_out), jnp.float32),
        scratch_types=[
            pltpu.VMEM((CHUNK, d_out), jnp.float32),
            pltpu.VMEM((CHUNK, LATENT_CODE_DIM), jnp.float32),
            pltpu.VMEM((CHUNK, PAIR), jnp.float32),
            pltpu.VMEM((CHUNK, PAIR), jnp.float32),
            pltpu.VMEM((CHUNK,), jnp.float32),
            pltpu.VMEM((CHUNK,), jnp.int32),
            pltpu.VMEM((CHUNK,), jnp.int32),
            pltpu.VMEM((CHUNK,), jnp.int32),
            pltpu.VMEM((CHUNK,), jnp.int32),
            pltpu.VMEM((CHUNK,), jnp.float32),
            pltpu.VMEM((CHUNK,), jnp.float32),
            pltpu.SemaphoreType.DMA,
            pltpu.SemaphoreType.DMA,
        ],
    )
    def body(x_hbm, xt_hbm, table_hbm, out_hbm, out_v, x_v, f_v, c_v, t_v,
             if_v, ic_v, of_v, oc_v, w1_v, w2_v, sem1, sem2):
        wid = lax.axis_index("s") * info.num_cores + lax.axis_index("c")
        base_w = wid * rows_per_w

        for ch in range(n_chunks):
            row0 = base_w + ch * CHUNK

            pltpu.sync_copy(x_hbm.at[pl.ds(row0, CHUNK), :], x_v)
            pltpu.sync_copy(xt_hbm.at[pl.ds(row0, CHUNK)], t_v)

            def phase_idx(g, carry):
                sl = pl.ds(g * LANES, LANES)
                t = t_v[sl]
                t = jnp.minimum(jnp.maximum(t, 0.0), 1.0)
                idx = t * scale
                fl = idx.astype(jnp.int32)
                flf = fl.astype(jnp.float32)
                cl = fl + jnp.sign(idx - flf).astype(jnp.int32)
                if_v[sl] = lax.shift_right_logical(fl, 1)
                ic_v[sl] = lax.shift_right_logical(cl, 1)
                of_v[sl] = (fl & 1) * LATENT_CODE_DIM
                oc_v[sl] = (cl & 1) * LATENT_CODE_DIM
                w1_v[sl] = 1.0 - (idx - flf)
                w2_v[sl] = flf
                return carry

            lax.fori_loop(0, n_groups, phase_idx, 0)

            cp_f = pltpu.async_copy(table_hbm.at[if_v], f_v, sem1)
            cp_c = pltpu.async_copy(table_hbm.at[ic_v], c_v, sem2)
            cp_f.wait()
            cp_c.wait()

            def phase_blend(g, carry):
                sl16 = pl.ds(g * LANES, LANES)
                w1g = w1_v[sl16]
                w2g = w2_v[sl16]
                ofg = of_v[sl16]
                ocg = oc_v[sl16]
                for j in range(LANES):
                    r = g * LANES + j
                    w1 = w1g[j]
                    w2 = w2g[j]
                    pof = ofg[j]
                    poc = ocg[j]
                    for k in range(LATENT_CODE_DIM // LANES):
                        sl = pl.ds(k * LANES, LANES)
                        out_v[r, sl] = x_v[r, sl]
                    for k in range(LATENT_CODE_DIM // LANES):
                        fsl = pl.ds(pof + k * LANES, LANES)
                        csl = pl.ds(poc + k * LANES, LANES)
                        osl = pl.ds(d_in - 1 + k * LANES, LANES)
                        out_v[r, osl] = w1 * f_v[r, fsl] + w2 * c_v[r, csl]
                return carry

            lax.fori_loop(0, n_groups, phase_blend, 0)

            pltpu.sync_copy(out_v, out_hbm.at[pl.ds(row0, CHUNK), :])

    return body


def kernel(x, latent_codes):
    batch, d_in = x.shape
    table_pairs = latent_codes.reshape(NUM_LATENT_CODES // 2, PAIR)
    return _build(batch, d_in)(x, x[:, -1], table_pairs)


# speedup vs baseline: 1.0490x; 1.0196x over previous
import functools

import jax
import jax.numpy as jnp
from jax import lax
from jax.experimental import pallas as pl
from jax.experimental.pallas import tpu as pltpu
from jax.experimental.pallas import tpu_sc as plsc

NUM_LATENT_CODES = 100000
LATENT_CODE_DIM = 64
LANES = 16
CHUNK = 128


@functools.lru_cache(maxsize=None)
def _build(batch, d_in):
    d_out = d_in - 1 + LATENT_CODE_DIM
    info = plsc.get_sparse_core_info()
    num_workers = info.num_cores * info.num_subcores
    rows_per_w = batch // num_workers
    n_chunks = rows_per_w // CHUNK
    n_groups = CHUNK // LANES
    scale = float(NUM_LATENT_CODES - 1)

    mesh = plsc.VectorSubcoreMesh(core_axis_name="c", subcore_axis_name="s")

    @functools.partial(
        pl.kernel,
        mesh=mesh,
        compiler_params=pltpu.CompilerParams(use_tc_tiling_on_sc=False),
        out_type=jax.ShapeDtypeStruct((batch, d_out), jnp.float32),
        scratch_types=[
            pltpu.VMEM((CHUNK, d_out), jnp.float32),
            pltpu.VMEM((CHUNK, LATENT_CODE_DIM), jnp.float32),
            pltpu.VMEM((CHUNK, LATENT_CODE_DIM), jnp.float32),
            pltpu.VMEM((CHUNK,), jnp.float32),
            pltpu.VMEM((CHUNK,), jnp.int32),
            pltpu.VMEM((CHUNK,), jnp.int32),
            pltpu.VMEM((CHUNK,), jnp.float32),
            pltpu.VMEM((CHUNK,), jnp.float32),
            pltpu.SemaphoreType.DMA,
            pltpu.SemaphoreType.DMA,
        ],
    )
    def body(x_hbm, xt_hbm, table_hbm, out_hbm, out_v, f_v, c_v, t_v, if_v,
             ic_v, w1_v, w2_v, sem1, sem2):
        wid = lax.axis_index("s") * info.num_cores + lax.axis_index("c")
        base_w = wid * rows_per_w

        for ch in range(n_chunks):
            row0 = base_w + ch * CHUNK

            pltpu.sync_copy(
                x_hbm.at[pl.ds(row0, CHUNK), :],
                out_v.at[:, pl.ds(0, d_in)],
            )
            pltpu.sync_copy(xt_hbm.at[pl.ds(row0, CHUNK)], t_v)

            def phase_idx(g, carry):
                sl = pl.ds(g * LANES, LANES)
                t = t_v[sl]
                t = jnp.minimum(jnp.maximum(t, 0.0), 1.0)
                idx = t * scale
                fl = idx.astype(jnp.int32)
                flf = fl.astype(jnp.float32)
                cl = fl + jnp.sign(idx - flf).astype(jnp.int32)
                if_v[sl] = fl
                ic_v[sl] = cl
                w1_v[sl] = 1.0 - (idx - flf)
                w2_v[sl] = flf
                return carry

            lax.fori_loop(0, n_groups, phase_idx, 0)

            cp_f = pltpu.async_copy(table_hbm.at[if_v], f_v, sem1)
            cp_c = pltpu.async_copy(table_hbm.at[ic_v], c_v, sem2)
            cp_f.wait()
            cp_c.wait()

            def phase_blend(g, carry):
                sl16 = pl.ds(g * LANES, LANES)
                w1g = w1_v[sl16]
                w2g = w2_v[sl16]
                for j in range(LANES):
                    r = g * LANES + j
                    w1 = w1g[j]
                    w2 = w2g[j]
                    for k in range(LATENT_CODE_DIM // LANES):
                        sl = pl.ds(k * LANES, LANES)
                        osl = pl.ds(d_in - 1 + k * LANES, LANES)
                        out_v[r, osl] = w1 * f_v[r, sl] + w2 * c_v[r, sl]
                return carry

            lax.fori_loop(0, n_groups, phase_blend, 0)

            pltpu.sync_copy(out_v, out_hbm.at[pl.ds(row0, CHUNK), :])

    return body


def kernel(x, latent_codes):
    batch, d_in = x.shape
    return _build(batch, d_in)(x, x[:, -1], latent_codes)
